# baseline (speedup 1.0000x reference)
import functools

import jax
import jax.numpy as jnp
from jax import lax
from jax.experimental import pallas as pl
from jax.experimental.pallas import tpu as pltpu
from jax.experimental.pallas import tpu_sc as plsc

N = 10000
E = 320000
NW = 32
EW = 10000
NCH = 79
CH = 128
NPS = 624

_MESH = plsc.VectorSubcoreMesh(core_axis_name="c", subcore_axis_name="s")


def _ale_body(ea_ref, s_ref, o0_ref, o1_ref, o2_ref):
    o = jnp.dot(ea_ref[...], s_ref[...], preferred_element_type=jnp.float32)
    o0_ref[...] = o[:, 0:8]
    o1_ref[...] = o[:, 8:16]
    o2_ref[...] = o[:, 16:24]


def _ale_all(ea2, S3):
    return pl.pallas_call(
        _ale_body,
        grid=(10,),
        in_specs=[pl.BlockSpec((4000, 128), lambda i: (i, 0)),
                  pl.BlockSpec((128, 24), lambda i: (0, 0))],
        out_specs=[pl.BlockSpec((4000, 8), lambda i: (i, 0))] * 3,
        out_shape=[jax.ShapeDtypeStruct((E // 8, 8), jnp.float32)] * 3,
    )(ea2, S3)


def _interleave(h):
    hs = jnp.concatenate([h[:, 64:], h[:, :64]], axis=1)
    return jnp.stack([h, hs], axis=1).reshape(2 * h.shape[0], 128)


def _tc0_body(x_ref, w_ref, a_ref, ht_ref, al_ref):
    h = jnp.dot(x_ref[...], w_ref[...], preferred_element_type=jnp.float32)
    ht_ref[...] = _interleave(h)
    al_ref[...] = jnp.dot(h, a_ref[...], preferred_element_type=jnp.float32)


def _tc0(x, W, A):
    return pl.pallas_call(
        _tc0_body,
        grid=(10,),
        in_specs=[pl.BlockSpec((1000, 128), lambda i: (i, 0)),
                  pl.BlockSpec((128, 128), lambda i: (0, 0)),
                  pl.BlockSpec((128, 2), lambda i: (0, 0))],
        out_specs=[pl.BlockSpec((2000, 128), lambda i: (i, 0)),
                   pl.BlockSpec((1000, 2), lambda i: (i, 0))],
        out_shape=[jax.ShapeDtypeStruct((2 * N, 128), jnp.float32),
                   jax.ShapeDtypeStruct((N, 2), jnp.float32)],
    )(x, W, A)


def _tcl_body(o0_ref, o1_ref, d0_ref, d1_ref, b_ref, w_ref, a_ref,
              ht_ref, al_ref):
    inv = 1.0 / (d0_ref[...] + d1_ref[...] + 1e-16)
    act = jax.nn.relu((o0_ref[...] + o1_ref[...]) * inv + b_ref[...])
    h = jnp.dot(act, w_ref[...], preferred_element_type=jnp.float32)
    ht_ref[...] = _interleave(h)
    al_ref[...] = jnp.dot(h, a_ref[...], preferred_element_type=jnp.float32)


def _tcl(o0, o1, d0, d1, b, W, A):
    return pl.pallas_call(
        _tcl_body,
        grid=(10,),
        in_specs=[pl.BlockSpec((1000, 64), lambda i: (i, 0)),
                  pl.BlockSpec((1000, 64), lambda i: (i, 0)),
                  pl.BlockSpec((1000, 1), lambda i: (i, 0)),
                  pl.BlockSpec((1000, 1), lambda i: (i, 0)),
                  pl.BlockSpec((1, 64), lambda i: (0, 0)),
                  pl.BlockSpec((64, 128), lambda i: (0, 0)),
                  pl.BlockSpec((128, 2), lambda i: (0, 0))],
        out_specs=[pl.BlockSpec((2000, 128), lambda i: (i, 0)),
                   pl.BlockSpec((1000, 2), lambda i: (i, 0))],
        out_shape=[jax.ShapeDtypeStruct((2 * N, 128), jnp.float32),
                   jax.ShapeDtypeStruct((N, 2), jnp.float32)],
    )(o0, o1, d0, d1, b, W, A)


def _fin_body(o0_ref, o1_ref, d0_ref, d1_ref, b_ref, w_ref, bfc_ref, y_ref):
    inv = 1.0 / (d0_ref[...] + d1_ref[...] + 1e-16)
    act = jax.nn.relu((o0_ref[...] + o1_ref[...]) * inv + b_ref[...])
    y_ref[...] = jax.nn.relu(
        jnp.dot(act, w_ref[...], preferred_element_type=jnp.float32) + bfc_ref[...])


def _tcfin(o0, o1, d0, d1, b, Wfc, bfc):
    return pl.pallas_call(
        _fin_body,
        grid=(10,),
        in_specs=[pl.BlockSpec((1000, 64), lambda i: (i, 0)),
                  pl.BlockSpec((1000, 64), lambda i: (i, 0)),
                  pl.BlockSpec((1000, 1), lambda i: (i, 0)),
                  pl.BlockSpec((1000, 1), lambda i: (i, 0)),
                  pl.BlockSpec((1, 64), lambda i: (0, 0)),
                  pl.BlockSpec((64, 512), lambda i: (0, 0)),
                  pl.BlockSpec((1, 512), lambda i: (0, 0))],
        out_specs=pl.BlockSpec((1000, 512), lambda i: (i, 0)),
        out_shape=jax.ShapeDtypeStruct((N, 512), jnp.float32),
    )(o0, o1, d0, d1, b, Wfc, bfc)


@functools.partial(
    pl.kernel,
    out_type=[jax.ShapeDtypeStruct((NW, NCH, CH), jnp.float32),
              jax.ShapeDtypeStruct((NW, 16), jnp.float32)],
    mesh=_MESH,
    compiler_params=pltpu.CompilerParams(needs_layout_passes=False),
    scratch_types=[pltpu.VMEM((NCH, CH), jnp.int32),
                   pltpu.VMEM((NCH, CH), jnp.int32),
                   pltpu.VMEM((NCH, CH), jnp.float32),
                   pltpu.VMEM((N,), jnp.float32),
                   pltpu.VMEM((N,), jnp.float32),
                   pltpu.VMEM((NCH, CH), jnp.float32),
                   pltpu.VMEM((16,), jnp.float32)],
)
def _sc1(src_h, dst_h, ale_h, als_h, ald_h, alpha_h, maxes_h,
         src_v, dst_v, ale_v, als_v, ald_v, alpha_v, mx_v):
    c = lax.axis_index("c")
    s = lax.axis_index("s")
    w = c * 16 + s
    pltpu.sync_copy(src_h.at[w], src_v)
    pltpu.sync_copy(dst_h.at[w], dst_v)
    pltpu.sync_copy(ale_h.at[w], ale_v)
    pltpu.sync_copy(als_h, als_v)
    pltpu.sync_copy(ald_h, ald_v)

    def body(j, m):
        for k in range(CH // 16):
            si = src_v[j, pl.ds(16 * k, 16)]
            di = dst_v[j, pl.ds(16 * k, 16)]
            a = (plsc.load_gather(als_v, [si])
                 + plsc.load_gather(ald_v, [di])
                 + ale_v[j, pl.ds(16 * k, 16)])
            a = jnp.maximum(a, 0.0) + 0.2 * jnp.minimum(a, 0.0)
            m = jnp.maximum(m, a)
            alpha_v[j, pl.ds(16 * k, 16)] = a
        return m

    m = lax.fori_loop(0, NCH, body, jnp.full((16,), -1e30, jnp.float32))

    mx_v[...] = m
    pltpu.sync_copy(alpha_v, alpha_h.at[w])
    pltpu.sync_copy(mx_v, maxes_h.at[w])


@functools.partial(
    pl.kernel,
    out_type=[jax.ShapeDtypeStruct((2 * N,), jnp.float32),
              jax.ShapeDtypeStruct((2, N // 2, 128), jnp.float32)],
    mesh=_MESH,
    compiler_params=pltpu.CompilerParams(needs_layout_passes=False),
    scratch_types=[pltpu.VMEM((NCH, CH), jnp.float32),
                   pltpu.VMEM((NCH, CH), jnp.int32),
                   pltpu.VMEM((NCH, CH), jnp.int32),
                   pltpu.VMEM((NW, 16), jnp.float32),
                   pltpu.VMEM((640,), jnp.float32),
                   [pltpu.VMEM((CH,), jnp.float32)] * 2,
                   [pltpu.VMEM((CH,), jnp.int32)] * 2,
                   [pltpu.VMEM((CH,), jnp.int32)] * 2,
                   [pltpu.VMEM((CH,), jnp.int32)] * 2,
                   [pltpu.VMEM((CH, 128), jnp.float32)] * 2,
                   pltpu.VMEM_SHARED((N,), jnp.float32),
                   pltpu.VMEM_SHARED((N // 2, 128), jnp.float32),
                   [pltpu.SemaphoreType.DMA] * 6],
)
def _sc2(alpha_h, maxes_h, src_h, dst_h, h_h, den_h, out_h,
         alpha_v, src_v, dst_v, mx_v, dstage_v, exs, igs, iss,
         idds, rowss, den_acc, out_acc, sems):
    c = lax.axis_index("c")
    s = lax.axis_index("s")
    w = c * 16 + s
    r0 = s * NPS
    gsems = sems[0:2]
    ssems = sems[2:4]
    esems = sems[4:6]

    z16 = jnp.zeros((16,), jnp.float32)

    zi16 = jnp.zeros((16,), jnp.int32)

    def zrow(i, carry):
        for q in range(8):
            rowss[0][i, pl.ds(16 * q, 16)] = z16
            rowss[1][i, pl.ds(16 * q, 16)] = z16
        return carry

    lax.fori_loop(0, CH, zrow, 0)
    for t in range(CH // 16):
        exs[1][pl.ds(16 * t, 16)] = z16
        iss[1][pl.ds(16 * t, 16)] = zi16
        idds[1][pl.ds(16 * t, 16)] = zi16
    for t in range(40):
        dstage_v[pl.ds(16 * t, 16)] = z16
    pltpu.sync_copy(dstage_v, den_acc.at[pl.ds(r0, 640)])
    q0 = s * 312
    for t in range(2):
        pltpu.sync_copy(rowss[0], out_acc.at[pl.ds(q0 + 128 * t, 128)])
    pltpu.sync_copy(rowss[0].at[pl.ds(0, 64)], out_acc.at[pl.ds(q0 + 256, 64)])

    pltpu.sync_copy(alpha_h.at[w], alpha_v)
    pltpu.sync_copy(src_h.at[w], src_v)
    pltpu.sync_copy(dst_h.at[w], dst_v)
    pltpu.sync_copy(maxes_h, mx_v)
    plsc.subcore_barrier()

    m = mx_v[0, :]
    for i in range(1, NW):
        m = jnp.maximum(m, mx_v[i, :])
    g = jnp.max(m)
    gv = lax.broadcast_in_dim(g, (16,), ())

    def prep(jn, b):
        for k in range(CH // 16):
            si = src_v[jn, pl.ds(16 * k, 16)]
            di = dst_v[jn, pl.ds(16 * k, 16)]
            a = alpha_v[jn, pl.ds(16 * k, 16)]
            igs[b][pl.ds(16 * k, 16)] = si * 2 + (di & 1)
            iss[b][pl.ds(16 * k, 16)] = di >> 1
            idds[b][pl.ds(16 * k, 16)] = di
            exs[b][pl.ds(16 * k, 16)] = jnp.exp(a - gv)
        pltpu.async_copy(h_h.at[igs[b]], rowss[b], gsems[b])

    def scale(b):
        def sk(k, carry):
            exk = exs[b][pl.ds(16 * k, 16)]
            for i in range(16):
                cv = lax.broadcast_in_dim(exk[i], (16,), ())
                for q in range(8):
                    sl = pl.ds(16 * q, 16)
                    rowss[b][16 * k + i, sl] = rowss[b][16 * k + i, sl] * cv
            return carry
        lax.fori_loop(0, CH // 16, sk, 0)

    def wait_gather(b):
        pltpu.make_async_copy(h_h.at[igs[b]], rowss[b], gsems[b]).wait()

    def launch_scatter(b):
        pltpu.async_copy(exs[b], den_acc.at[idds[b]], esems[b], add=True)
        pltpu.async_copy(rowss[b], out_acc.at[iss[b]], ssems[b], add=True)

    def wait_scatter(b):
        pltpu.make_async_copy(exs[b], den_acc.at[idds[b]], esems[b]).wait()
        pltpu.make_async_copy(rowss[b], out_acc.at[iss[b]], ssems[b]).wait()


    launch_scatter(1)
    prep(0, 0)

    def pair(jj, carry):
        wait_gather(0)
        wait_scatter(1)
        prep(2 * jj + 1, 1)
        scale(0)
        launch_scatter(0)
        wait_gather(1)
        wait_scatter(0)
        prep(2 * jj + 2, 0)
        scale(1)
        launch_scatter(1)
        return carry

    lax.fori_loop(0, (NCH - 1) // 2, pair, 0)
    wait_gather(0)
    wait_scatter(1)
    scale(0)
    launch_scatter(0)
    wait_scatter(0)
    plsc.subcore_barrier()

    pltpu.sync_copy(den_acc.at[pl.ds(r0, NPS)], dstage_v.at[pl.ds(0, NPS)])
    pltpu.sync_copy(dstage_v.at[pl.ds(0, NPS)], den_h.at[pl.ds(c * N + r0, NPS)])
    for t in range(2):
        pltpu.sync_copy(out_acc.at[pl.ds(q0 + 128 * t, 128)], rowss[0])
        pltpu.sync_copy(rowss[0], out_h.at[c, pl.ds(q0 + 128 * t, 128)])
    pltpu.sync_copy(out_acc.at[pl.ds(q0 + 256, 56)], rowss[0].at[pl.ds(0, 56)])
    pltpu.sync_copy(rowss[0].at[pl.ds(0, 56)], out_h.at[c, pl.ds(q0 + 256, 56)])

    @pl.when(s == 15)
    def _():
        t0 = 16 * NPS
        pltpu.sync_copy(den_acc.at[pl.ds(t0, 16)], dstage_v.at[pl.ds(0, 16)])
        pltpu.sync_copy(dstage_v.at[pl.ds(0, 16)], den_h.at[pl.ds(c * N + t0, 16)])
        q1 = 16 * 312
        pltpu.sync_copy(out_acc.at[pl.ds(q1, 8)], rowss[0].at[pl.ds(0, 8)])
        pltpu.sync_copy(rowss[0].at[pl.ds(0, 8)], out_h.at[c, pl.ds(q1, 8)])


def kernel(x, edge_index, edge_attr, W0, as0, ad0, We0, ae0, b0,
           W1, as1, ad1, We1, ae1, b1, W2, as2, ad2, We2, ae2, b2, Wfc, bfc):
    f32 = jnp.float32
    npad = NCH * CH - EW
    src3 = jnp.pad(edge_index[0].reshape(NW, EW),
                   ((0, 0), (0, npad))).reshape(NW, NCH, CH)
    dst3 = jnp.pad(edge_index[1].reshape(NW, EW),
                   ((0, 0), (0, npad))).reshape(NW, NCH, CH)
    ea2 = edge_attr.reshape(E // 8, 128)

    wv = jnp.stack([We0 @ ae0[0], We1 @ ae1[0], We2 @ ae2[0]], 0)
    S3 = (jnp.eye(8, dtype=f32)[:, None, None, :]
          * wv.T[None, :, :, None]).reshape(128, 24)
    ale3 = [jnp.pad(a.reshape(NW, EW), ((0, 0), (0, npad)),
                    constant_values=-1e30).reshape(NW, NCH, CH)
            for a in _ale_all(ea2, S3)]

    As = [jnp.pad(jnp.stack([as0[0], ad0[0]], axis=1), ((0, 64), (0, 0))),
          jnp.pad(jnp.stack([as1[0], ad1[0]], axis=1), ((0, 64), (0, 0))),
          jnp.pad(jnp.stack([as2[0], ad2[0]], axis=1), ((0, 64), (0, 0)))]
    Ws = [jnp.pad(W0, ((0, 0), (0, 64))),
          jnp.pad(W1, ((0, 0), (0, 64))),
          jnp.pad(W2, ((0, 0), (0, 64)))]
    bs = [b0, b1, b2]

    hT, al = _tc0(x, Ws[0], As[0])
    for l in range(3):
        als = al[:, 0]
        ald = al[:, 1]
        alpha, maxes = _sc1(src3, dst3, ale3[l], als, ald)
        den, out = _sc2(alpha, maxes, src3, dst3, hT)
        den = den.reshape(2, N)
        d0 = den[0].reshape(N, 1)
        d1 = den[1].reshape(N, 1)
        o = out.reshape(2, N, 64)
        o0 = o[0]
        o1 = o[1]
        if l < 2:
            hT, al = _tcl(o0, o1, d0, d1, bs[l].reshape(1, 64),
                          Ws[l + 1], As[l + 1])
    return _tcfin(o0, o1, d0, d1, bs[2].reshape(1, 64),
                  Wfc, bfc.reshape(1, 512))


# speedup vs baseline: 1.3643x; 1.3643x over previous
import functools

import jax
import jax.numpy as jnp
from jax import lax
from jax.experimental import pallas as pl
from jax.experimental.pallas import tpu as pltpu
from jax.experimental.pallas import tpu_sc as plsc

N = 10000
E = 320000
NW = 32
EW = 10000
NCH = 125
CH = 80
NPS = 624

_MESH = plsc.VectorSubcoreMesh(core_axis_name="c", subcore_axis_name="s")


def _ale_body(ea_ref, s_ref, o0_ref, o1_ref, o2_ref):
    o = jnp.dot(ea_ref[...], s_ref[...], preferred_element_type=jnp.float32)
    o0_ref[...] = o[:, 0:8]
    o1_ref[...] = o[:, 8:16]
    o2_ref[...] = o[:, 16:24]


def _ale_all(ea2, S3):
    return pl.pallas_call(
        _ale_body,
        grid=(10,),
        in_specs=[pl.BlockSpec((4000, 128), lambda i: (i, 0)),
                  pl.BlockSpec((128, 24), lambda i: (0, 0))],
        out_specs=[pl.BlockSpec((4000, 8), lambda i: (i, 0))] * 3,
        out_shape=[jax.ShapeDtypeStruct((E // 8, 8), jnp.float32)] * 3,
    )(ea2, S3)


def _interleave(h):
    hs = jnp.concatenate([h[:, 64:], h[:, :64]], axis=1)
    return jnp.stack([h, hs], axis=1).reshape(2 * h.shape[0], 128)


def _tc0_body(x_ref, w_ref, a_ref, ht_ref, al_ref):
    h = jnp.dot(x_ref[...], w_ref[...], preferred_element_type=jnp.float32)
    ht_ref[...] = _interleave(h)
    al_ref[...] = jnp.dot(h, a_ref[...], preferred_element_type=jnp.float32)


def _tc0(x, W, A):
    return pl.pallas_call(
        _tc0_body,
        grid=(10,),
        in_specs=[pl.BlockSpec((1000, 128), lambda i: (i, 0)),
                  pl.BlockSpec((128, 128), lambda i: (0, 0)),
                  pl.BlockSpec((128, 2), lambda i: (0, 0))],
        out_specs=[pl.BlockSpec((2000, 128), lambda i: (i, 0)),
                   pl.BlockSpec((1000, 2), lambda i: (i, 0))],
        out_shape=[jax.ShapeDtypeStruct((2 * N, 128), jnp.float32),
                   jax.ShapeDtypeStruct((N, 2), jnp.float32)],
    )(x, W, A)


def _tcl_body(o0_ref, o1_ref, d0_ref, d1_ref, b_ref, w_ref, a_ref,
              ht_ref, al_ref):
    inv = 1.0 / (d0_ref[...] + d1_ref[...] + 1e-16)
    act = jax.nn.relu((o0_ref[...] + o1_ref[...]) * inv + b_ref[...])
    h = jnp.dot(act, w_ref[...], preferred_element_type=jnp.float32)
    ht_ref[...] = _interleave(h)
    al_ref[...] = jnp.dot(h, a_ref[...], preferred_element_type=jnp.float32)


def _tcl(o0, o1, d0, d1, b, W, A):
    return pl.pallas_call(
        _tcl_body,
        grid=(10,),
        in_specs=[pl.BlockSpec((1000, 64), lambda i: (i, 0)),
                  pl.BlockSpec((1000, 64), lambda i: (i, 0)),
                  pl.BlockSpec((1000, 1), lambda i: (i, 0)),
                  pl.BlockSpec((1000, 1), lambda i: (i, 0)),
                  pl.BlockSpec((1, 64), lambda i: (0, 0)),
                  pl.BlockSpec((64, 128), lambda i: (0, 0)),
                  pl.BlockSpec((128, 2), lambda i: (0, 0))],
        out_specs=[pl.BlockSpec((2000, 128), lambda i: (i, 0)),
                   pl.BlockSpec((1000, 2), lambda i: (i, 0))],
        out_shape=[jax.ShapeDtypeStruct((2 * N, 128), jnp.float32),
                   jax.ShapeDtypeStruct((N, 2), jnp.float32)],
    )(o0, o1, d0, d1, b, W, A)


def _fin_body(o0_ref, o1_ref, d0_ref, d1_ref, b_ref, w_ref, bfc_ref, y_ref):
    inv = 1.0 / (d0_ref[...] + d1_ref[...] + 1e-16)
    act = jax.nn.relu((o0_ref[...] + o1_ref[...]) * inv + b_ref[...])
    y_ref[...] = jax.nn.relu(
        jnp.dot(act, w_ref[...], preferred_element_type=jnp.float32) + bfc_ref[...])


def _tcfin(o0, o1, d0, d1, b, Wfc, bfc):
    return pl.pallas_call(
        _fin_body,
        grid=(10,),
        in_specs=[pl.BlockSpec((1000, 64), lambda i: (i, 0)),
                  pl.BlockSpec((1000, 64), lambda i: (i, 0)),
                  pl.BlockSpec((1000, 1), lambda i: (i, 0)),
                  pl.BlockSpec((1000, 1), lambda i: (i, 0)),
                  pl.BlockSpec((1, 64), lambda i: (0, 0)),
                  pl.BlockSpec((64, 512), lambda i: (0, 0)),
                  pl.BlockSpec((1, 512), lambda i: (0, 0))],
        out_specs=pl.BlockSpec((1000, 512), lambda i: (i, 0)),
        out_shape=jax.ShapeDtypeStruct((N, 512), jnp.float32),
    )(o0, o1, d0, d1, b, Wfc, bfc)


@functools.partial(
    pl.kernel,
    out_type=[jax.ShapeDtypeStruct((NW, NCH, CH), jnp.float32),
              jax.ShapeDtypeStruct((NW, 16), jnp.float32)],
    mesh=_MESH,
    compiler_params=pltpu.CompilerParams(needs_layout_passes=False),
    scratch_types=[pltpu.VMEM((NCH, CH), jnp.int32),
                   pltpu.VMEM((NCH, CH), jnp.int32),
                   pltpu.VMEM((NCH, CH), jnp.float32),
                   pltpu.VMEM((N,), jnp.float32),
                   pltpu.VMEM((N,), jnp.float32),
                   pltpu.VMEM((NCH, CH), jnp.float32),
                   pltpu.VMEM((16,), jnp.float32)],
)
def _sc1(src_h, dst_h, ale_h, als_h, ald_h, alpha_h, maxes_h,
         src_v, dst_v, ale_v, als_v, ald_v, alpha_v, mx_v):
    c = lax.axis_index("c")
    s = lax.axis_index("s")
    w = c * 16 + s
    pltpu.sync_copy(src_h.at[w], src_v)
    pltpu.sync_copy(dst_h.at[w], dst_v)
    pltpu.sync_copy(ale_h.at[w], ale_v)
    pltpu.sync_copy(als_h, als_v)
    pltpu.sync_copy(ald_h, ald_v)

    def body(j, m):
        for k in range(CH // 16):
            si = src_v[j, pl.ds(16 * k, 16)]
            di = dst_v[j, pl.ds(16 * k, 16)]
            a = (plsc.load_gather(als_v, [si])
                 + plsc.load_gather(ald_v, [di])
                 + ale_v[j, pl.ds(16 * k, 16)])
            a = jnp.maximum(a, 0.0) + 0.2 * jnp.minimum(a, 0.0)
            m = jnp.maximum(m, a)
            alpha_v[j, pl.ds(16 * k, 16)] = a
        return m

    m = lax.fori_loop(0, NCH, body, jnp.full((16,), -1e30, jnp.float32))

    mx_v[...] = m
    pltpu.sync_copy(alpha_v, alpha_h.at[w])
    pltpu.sync_copy(mx_v, maxes_h.at[w])


@functools.partial(
    pl.kernel,
    out_type=[jax.ShapeDtypeStruct((2 * N,), jnp.float32),
              jax.ShapeDtypeStruct((2, N // 2, 128), jnp.float32)],
    mesh=_MESH,
    compiler_params=pltpu.CompilerParams(needs_layout_passes=False),
    scratch_types=[pltpu.VMEM((NCH, CH), jnp.float32),
                   pltpu.VMEM((NCH, CH), jnp.int32),
                   pltpu.VMEM((NCH, CH), jnp.int32),
                   pltpu.VMEM((NW, 16), jnp.float32),
                   pltpu.VMEM((640,), jnp.float32),
                   [pltpu.VMEM((CH,), jnp.float32)] * 2,
                   [pltpu.VMEM((CH,), jnp.int32)] * 2,
                   [pltpu.VMEM((CH,), jnp.int32)] * 2,
                   [pltpu.VMEM((CH,), jnp.int32)] * 2,
                   [pltpu.VMEM((CH, 128), jnp.float32)] * 2,
                   pltpu.VMEM_SHARED((N,), jnp.float32),
                   pltpu.VMEM_SHARED((N // 2, 128), jnp.float32),
                   [pltpu.SemaphoreType.DMA] * 6],
)
def _sc2(alpha_h, maxes_h, src_h, dst_h, h_h, den_h, out_h,
         alpha_v, src_v, dst_v, mx_v, dstage_v, exs, igs, iss,
         idds, rowss, den_acc, out_acc, sems):
    c = lax.axis_index("c")
    s = lax.axis_index("s")
    w = c * 16 + s
    r0 = s * NPS
    gsems = sems[0:2]
    ssems = sems[2:4]
    esems = sems[4:6]

    z16 = jnp.zeros((16,), jnp.float32)

    zi16 = jnp.zeros((16,), jnp.int32)

    def zrow(i, carry):
        for q in range(8):
            rowss[0][i, pl.ds(16 * q, 16)] = z16
            rowss[1][i, pl.ds(16 * q, 16)] = z16
        return carry

    lax.fori_loop(0, CH, zrow, 0)
    for t in range(CH // 16):
        exs[1][pl.ds(16 * t, 16)] = z16
        iss[1][pl.ds(16 * t, 16)] = zi16
        idds[1][pl.ds(16 * t, 16)] = zi16
    for t in range(40):
        dstage_v[pl.ds(16 * t, 16)] = z16
    pltpu.sync_copy(dstage_v, den_acc.at[pl.ds(r0, 640)])
    q0 = s * 312
    for t in range(4):
        pltpu.sync_copy(rowss[0], out_acc.at[pl.ds(q0 + 80 * t, 80)])

    pltpu.sync_copy(alpha_h.at[w], alpha_v)
    pltpu.sync_copy(src_h.at[w], src_v)
    pltpu.sync_copy(dst_h.at[w], dst_v)
    pltpu.sync_copy(maxes_h, mx_v)
    plsc.subcore_barrier()

    m = mx_v[0, :]
    for i in range(1, NW):
        m = jnp.maximum(m, mx_v[i, :])
    g = jnp.max(m)
    gv = lax.broadcast_in_dim(g, (16,), ())

    def prep(jn, b):
        for k in range(CH // 16):
            si = src_v[jn, pl.ds(16 * k, 16)]
            di = dst_v[jn, pl.ds(16 * k, 16)]
            a = alpha_v[jn, pl.ds(16 * k, 16)]
            igs[b][pl.ds(16 * k, 16)] = si * 2 + (di & 1)
            iss[b][pl.ds(16 * k, 16)] = di >> 1
            idds[b][pl.ds(16 * k, 16)] = di
            exs[b][pl.ds(16 * k, 16)] = jnp.exp(a - gv)
        pltpu.async_copy(h_h.at[igs[b]], rowss[b], gsems[b])

    def scale(b):
        def sk(k, carry):
            exk = exs[b][pl.ds(16 * k, 16)]
            for i in range(16):
                cv = lax.broadcast_in_dim(exk[i], (16,), ())
                for q in range(8):
                    sl = pl.ds(16 * q, 16)
                    rowss[b][16 * k + i, sl] = rowss[b][16 * k + i, sl] * cv
            return carry
        lax.fori_loop(0, CH // 16, sk, 0)

    def wait_gather(b):
        pltpu.make_async_copy(h_h.at[igs[b]], rowss[b], gsems[b]).wait()

    def launch_scatter(b):
        pltpu.async_copy(exs[b], den_acc.at[idds[b]], esems[b], add=True)
        pltpu.async_copy(rowss[b], out_acc.at[iss[b]], ssems[b], add=True)

    def wait_scatter(b):
        pltpu.make_async_copy(exs[b], den_acc.at[idds[b]], esems[b]).wait()
        pltpu.make_async_copy(rowss[b], out_acc.at[iss[b]], ssems[b]).wait()


    launch_scatter(1)
    prep(0, 0)

    def pair(jj, carry):
        wait_gather(0)
        wait_scatter(1)
        prep(2 * jj + 1, 1)
        scale(0)
        launch_scatter(0)
        wait_gather(1)
        wait_scatter(0)
        prep(2 * jj + 2, 0)
        scale(1)
        launch_scatter(1)
        return carry

    lax.fori_loop(0, (NCH - 1) // 2, pair, 0)
    wait_gather(0)
    wait_scatter(1)
    scale(0)
    launch_scatter(0)
    wait_scatter(0)
    plsc.subcore_barrier()

    pltpu.sync_copy(den_acc.at[pl.ds(r0, NPS)], dstage_v.at[pl.ds(0, NPS)])
    pltpu.sync_copy(dstage_v.at[pl.ds(0, NPS)], den_h.at[pl.ds(c * N + r0, NPS)])
    for t in range(3):
        pltpu.sync_copy(out_acc.at[pl.ds(q0 + 80 * t, 80)], rowss[0])
        pltpu.sync_copy(rowss[0], out_h.at[c, pl.ds(q0 + 80 * t, 80)])
    pltpu.sync_copy(out_acc.at[pl.ds(q0 + 240, 72)], rowss[0].at[pl.ds(0, 72)])
    pltpu.sync_copy(rowss[0].at[pl.ds(0, 72)], out_h.at[c, pl.ds(q0 + 240, 72)])

    @pl.when(s == 15)
    def _():
        t0 = 16 * NPS
        pltpu.sync_copy(den_acc.at[pl.ds(t0, 16)], dstage_v.at[pl.ds(0, 16)])
        pltpu.sync_copy(dstage_v.at[pl.ds(0, 16)], den_h.at[pl.ds(c * N + t0, 16)])
        q1 = 16 * 312
        pltpu.sync_copy(out_acc.at[pl.ds(q1, 8)], rowss[0].at[pl.ds(0, 8)])
        pltpu.sync_copy(rowss[0].at[pl.ds(0, 8)], out_h.at[c, pl.ds(q1, 8)])


def kernel(x, edge_index, edge_attr, W0, as0, ad0, We0, ae0, b0,
           W1, as1, ad1, We1, ae1, b1, W2, as2, ad2, We2, ae2, b2, Wfc, bfc):
    f32 = jnp.float32
    src3 = edge_index[0].reshape(NW, NCH, CH)
    dst3 = edge_index[1].reshape(NW, NCH, CH)
    ea2 = edge_attr.reshape(E // 8, 128)

    wv = jnp.stack([We0 @ ae0[0], We1 @ ae1[0], We2 @ ae2[0]], 0)
    S3 = (jnp.eye(8, dtype=f32)[:, None, None, :]
          * wv.T[None, :, :, None]).reshape(128, 24)
    ale3 = [a.reshape(NW, NCH, CH) for a in _ale_all(ea2, S3)]

    As = [jnp.pad(jnp.stack([as0[0], ad0[0]], axis=1), ((0, 64), (0, 0))),
          jnp.pad(jnp.stack([as1[0], ad1[0]], axis=1), ((0, 64), (0, 0))),
          jnp.pad(jnp.stack([as2[0], ad2[0]], axis=1), ((0, 64), (0, 0)))]
    Ws = [jnp.pad(W0, ((0, 0), (0, 64))),
          jnp.pad(W1, ((0, 0), (0, 64))),
          jnp.pad(W2, ((0, 0), (0, 64)))]
    bs = [b0, b1, b2]

    hT, al = _tc0(x, Ws[0], As[0])
    for l in range(3):
        als = al[:, 0]
        ald = al[:, 1]
        alpha, maxes = _sc1(src3, dst3, ale3[l], als, ald)
        den, out = _sc2(alpha, maxes, src3, dst3, hT)
        den = den.reshape(2, N)
        d0 = den[0].reshape(N, 1)
        d1 = den[1].reshape(N, 1)
        o = out.reshape(2, N, 64)
        o0 = o[0]
        o1 = o[1]
        if l < 2:
            hT, al = _tcl(o0, o1, d0, d1, bs[l].reshape(1, 64),
                          Ws[l + 1], As[l + 1])
    return _tcfin(o0, o1, d0, d1, bs[2].reshape(1, 64),
                  Wfc, bfc.reshape(1, 512))
